# baseline (speedup 1.0000x reference)
import functools

import jax
import jax.numpy as jnp
from jax import lax
from jax.experimental import pallas as pl
from jax.experimental.pallas import tpu as pltpu
from jax.experimental.pallas import tpu_sc as plsc

VOCAB = 100000
EMB_DIM = 64
BATCH = 16384

_info = plsc.get_sparse_core_info()
_NC = _info.num_cores
_NS = _info.num_subcores
_NW = _NC * _NS
_BPW = BATCH // _NW
_CHUNK = 128
_NCHUNK = _BPW // _CHUNK

_mesh = plsc.VectorSubcoreMesh(core_axis_name="c", subcore_axis_name="s")


@functools.partial(
    pl.kernel,
    mesh=_mesh,
    compiler_params=pltpu.CompilerParams(use_tc_tiling_on_sc=False),
    out_type=jax.ShapeDtypeStruct((BATCH, EMB_DIM), jnp.float32),
    scratch_types=[
        pltpu.VMEM((_BPW,), jnp.int32),
        pltpu.VMEM((_BPW, EMB_DIM), jnp.float32),
        pltpu.SemaphoreType.DMA,
        pltpu.SemaphoreType.DMA,
    ],
)
def _gather_kernel(idx_hbm, table_hbm, out_hbm, idx_v, rows_v, sem_g, sem_o):
    wid = lax.axis_index("s") * _NC + lax.axis_index("c")
    base = wid * _BPW
    pltpu.sync_copy(idx_hbm.at[pl.ds(base, _BPW)], idx_v)
    gathers = []
    for j in range(_NCHUNK):
        gathers.append(
            pltpu.async_copy(
                table_hbm.at[idx_v.at[pl.ds(j * _CHUNK, _CHUNK)]],
                rows_v.at[pl.ds(j * _CHUNK, _CHUNK)],
                sem_g,
            )
        )
    outs = []
    for j in range(_NCHUNK):
        gathers[j].wait()
        outs.append(
            pltpu.async_copy(
                rows_v.at[pl.ds(j * _CHUNK, _CHUNK)],
                out_hbm.at[pl.ds(base + j * _CHUNK, _CHUNK)],
                sem_o,
            )
        )
    for o in outs:
        o.wait()


def kernel(token_ids, embedding_weight):
    return _gather_kernel(token_ids.astype(jnp.int32), embedding_weight)


# speedup vs baseline: 1.4938x; 1.4938x over previous
import functools

import jax
import jax.numpy as jnp
from jax import lax
from jax.experimental import pallas as pl
from jax.experimental.pallas import tpu as pltpu
from jax.experimental.pallas import tpu_sc as plsc

VOCAB = 100000
EMB_DIM = 64
BATCH = 16384

_info = plsc.get_sparse_core_info()
_NC = _info.num_cores
_NS = _info.num_subcores
_NW = _NC * _NS
_BPW = BATCH // _NW

_mesh = plsc.VectorSubcoreMesh(core_axis_name="c", subcore_axis_name="s")


@functools.partial(
    pl.kernel,
    mesh=_mesh,
    compiler_params=pltpu.CompilerParams(use_tc_tiling_on_sc=True),
    out_type=jax.ShapeDtypeStruct((BATCH, EMB_DIM), jnp.float32),
    scratch_types=[
        pltpu.VMEM((_BPW,), jnp.int32),
        pltpu.VMEM((_BPW, EMB_DIM), jnp.float32),
        pltpu.SemaphoreType.DMA,
    ],
)
def _gather_kernel(idx_hbm, table_hbm, out_hbm, idx_v, rows_v, sem_g):
    wid = lax.axis_index("s") * _NC + lax.axis_index("c")
    base = wid * _BPW
    pltpu.sync_copy(idx_hbm.at[pl.ds(base, _BPW)], idx_v)

    def fire(g, carry):
        v16 = idx_v[pl.ds(g * 16, 16)]
        for j in range(16):
            pltpu.async_copy(
                table_hbm.at[pl.ds(v16[j], 1)],
                rows_v.at[pl.ds(g * 16 + j, 1)],
                sem_g,
            )
        return carry

    lax.fori_loop(0, _BPW // 16, fire, 0)
    def drain(i, carry):
        pltpu.make_async_copy(
            table_hbm.at[pl.ds(0, 1)],
            rows_v.at[pl.ds(0, 1)],
            sem_g,
        ).wait()
        return carry

    lax.fori_loop(0, _BPW, drain, 0, unroll=8)
    pltpu.sync_copy(rows_v, out_hbm.at[pl.ds(base, _BPW)])


def kernel(token_ids, embedding_weight):
    return _gather_kernel(token_ids.astype(jnp.int32), embedding_weight)
